# Initial kernel scaffold; baseline (speedup 1.0000x reference)
import jax
import jax.numpy as jnp
from jax.experimental import pallas as pl


def kernel(x_p, x_d, edge_attr_p, edge_attr_d, edge_index_p, edge_index_d, x_p_batch, x_d_batch, pool_W1, pool_b1, pool_W2, pool_b2, p_nn1_W1, p_nn1_b1, p_nn1_W2, p_nn1_b2, p_nn2_W1, p_nn2_b1, p_nn2_W2, p_nn2_b2, p_root0, p_bias0, p_root1, p_bias1, p_root2, p_bias2, d_nn1_W1, d_nn1_b1, d_nn1_W2, d_nn1_b2, d_nn2_W1, d_nn2_b1, d_nn2_W2, d_nn2_b2, d_root0, d_bias0, d_root1, d_bias1, d_root2, d_bias2, lin0_W, lin0_b, lin1_W, lin1_b):
    raise NotImplementedError("write your pallas kernel here")


# speedup vs baseline: 1.9357x; 1.9357x over previous
import jax
import jax.numpy as jnp
from jax import lax
from jax.experimental import pallas as pl
from jax.experimental.pallas import tpu as pltpu
from jax.experimental.pallas import tpu_sc as plsc

N = 10000
E = 160000
D = 128
DE = 16
H = 16
G = 64
N2 = 2 * N
E2 = 2 * E
KC = 17 * H
KCP = 384


_BE = 4000


def _h_body(ea_ref, lo_ref, w1a_ref, b1a_ref, w1b_ref, b1b_ref, h1_ref, h2_ref):
    ea = ea_ref[0]
    lo = lo_ref[0]
    i2 = lax.broadcasted_iota(jnp.int32, (_BE, 16), 1)
    pad = (i2 == lo).astype(jnp.float32)
    h1 = jnp.maximum(
        jnp.dot(ea, w1a_ref[0], preferred_element_type=jnp.float32) + b1a_ref[0], 0.0)
    h2 = jnp.maximum(
        jnp.dot(ea, w1b_ref[0], preferred_element_type=jnp.float32) + b1b_ref[0], 0.0)
    h1_ref[0] = jnp.concatenate([h1, pad], axis=1)
    h2_ref[0] = jnp.concatenate([h2, pad], axis=1)


def _edge_h(ea2, lo2, w1a, b1a, w1b, b1b):
    return pl.pallas_call(
        _h_body,
        grid=(2, E // _BE),
        in_specs=[
            pl.BlockSpec((1, _BE, DE), lambda s, b: (s, b, 0)),
            pl.BlockSpec((1, _BE, 1), lambda s, b: (s, b, 0)),
            pl.BlockSpec((1, DE, H), lambda s, b: (s, 0, 0)),
            pl.BlockSpec((1, 1, H), lambda s, b: (s, 0, 0)),
            pl.BlockSpec((1, DE, H), lambda s, b: (s, 0, 0)),
            pl.BlockSpec((1, 1, H), lambda s, b: (s, 0, 0)),
        ],
        out_specs=[
            pl.BlockSpec((1, _BE, 32), lambda s, b: (s, b, 0)),
            pl.BlockSpec((1, _BE, 32), lambda s, b: (s, b, 0)),
        ],
        out_shape=[jax.ShapeDtypeStruct((2, E, 32), jnp.float32)] * 2,
    )(ea2, lo2, w1a, b1a, w1b, b1b)


_BSC = 80
_PER_TILE = E2 // 32
_NBLK = _PER_TILE // _BSC
_AROWS = 5120
_ROWS_T = _AROWS // 16
_NCH = _ROWS_T // _BSC


def _edge_body(y_hbm, h_hbm, src_hbm, dsth_hbm, z_hbm, rid_hbm, out_hbm,
               src_v, dsth_v, dsth_v2, rid_v, rows_v, h_v, msg_v, msg_v2,
               zbuf_v, agg_sh, sem):
    c = lax.axis_index("c")
    s = lax.axis_index("s")
    wid = s * 2 + c
    pltpu.sync_copy(z_hbm, zbuf_v)
    pltpu.sync_copy(z_hbm, msg_v)
    pltpu.sync_copy(z_hbm, msg_v2)

    def zch(j, carry):
        pltpu.sync_copy(rid_hbm.at[pl.ds(s * _NCH + j, 1)], rid_v)
        pltpu.sync_copy(zbuf_v, agg_sh.at[rid_v.at[0]])
        return carry

    lax.fori_loop(0, _NCH, zch, 0)
    plsc.subcore_barrier()

    base = wid * _PER_TILE
    base_blk = wid * _NBLK

    def do_block(b, mbuf, dbuf):
        off = base + b * _BSC
        pltpu.sync_copy(src_hbm.at[pl.ds(off, _BSC)], src_v)
        pltpu.sync_copy(dsth_hbm.at[pl.ds(base_blk + b, 1)], dbuf)
        pltpu.sync_copy(h_hbm.at[pl.ds(off, _BSC)], h_v)
        pltpu.async_copy(y_hbm.at[src_v], rows_v, sem).wait()

        def e_body(e, c2):
            hrow = h_v[e, pl.ds(0, 16)]
            lrow = h_v[e, pl.ds(16, 16)]
            acc = rows_v[e, pl.ds(256, 16)]
            for k in range(16):
                acc = acc + hrow[k] * rows_v[e, pl.ds(k * 16, 16)]
            mbuf[e, pl.ds(0, 16)] = acc * lrow[0]
            mbuf[e, pl.ds(16, 16)] = acc * lrow[1]
            mbuf[e, pl.ds(32, 16)] = acc * lrow[2]
            mbuf[e, pl.ds(48, 16)] = acc * lrow[3]
            return c2

        lax.fori_loop(0, _BSC, e_body, 0)
        pltpu.sync_copy(mbuf, agg_sh.at[dbuf.at[0]], add=True)

    def blk2(j, carry):
        do_block(2 * j, msg_v, dsth_v)
        do_block(2 * j + 1, msg_v2, dsth_v2)
        return carry

    lax.fori_loop(0, _NBLK // 2, blk2, 0)
    do_block(_NBLK - 1, msg_v, dsth_v)
    plsc.subcore_barrier()

    def och(j, carry):
        pltpu.sync_copy(rid_hbm.at[pl.ds(s * _NCH + j, 1)], rid_v)
        pltpu.sync_copy(agg_sh.at[rid_v.at[0]], zbuf_v)
        pltpu.sync_copy(zbuf_v,
                        out_hbm.at[c, pl.ds(s * _ROWS_T + j * _BSC, _BSC)])
        return carry

    lax.fori_loop(0, _NCH, och, 0)


_edge_step = pl.kernel(
    _edge_body,
    mesh=plsc.VectorSubcoreMesh(core_axis_name="c", subcore_axis_name="s"),
    out_type=jax.ShapeDtypeStruct((2, _AROWS, 128), jnp.float32),
    scratch_types=[
        pltpu.VMEM((_BSC,), jnp.int32),
        pltpu.VMEM((1, _BSC), jnp.int32),
        pltpu.VMEM((1, _BSC), jnp.int32),
        pltpu.VMEM((1, _BSC), jnp.int32),
        pltpu.VMEM((_BSC, KCP), jnp.float32),
        pltpu.VMEM((_BSC, 32), jnp.float32),
        pltpu.VMEM((_BSC, 128), jnp.float32),
        pltpu.VMEM((_BSC, 128), jnp.float32),
        pltpu.VMEM((_BSC, 128), jnp.float32),
        pltpu.VMEM_SHARED((_AROWS, 128), jnp.float32),
        pltpu.SemaphoreType.DMA,
    ],
)


def _unpack(agg):
    u = (agg[0, :, :64] + agg[1, :, :64]).reshape(2, 2560 * 4, H)
    return u[:, :N]


_BN = 2000


def _layer(agg, x2, roots, biases, tmat, *, relu, emit_y):
    has_agg = agg is not None
    c = x2.shape[2]
    nb = N // _BN
    in_specs = []
    args = []
    if has_agg:
        in_specs.append(pl.BlockSpec((1, _BN, H), lambda s, b: (s, b, 0)))
        args.append(agg)
    in_specs.append(pl.BlockSpec((1, _BN, c), lambda s, b: (s, b, 0)))
    args.append(x2)
    if has_agg:
        in_specs += [
            pl.BlockSpec((1, c, H), lambda s, b: (s, 0, 0)),
            pl.BlockSpec((1, 1, H), lambda s, b: (s, 0, 0)),
        ]
        args += [roots, biases]
    if emit_y:
        tc_in = H if has_agg else c
        in_specs.append(pl.BlockSpec((1, tc_in, KCP), lambda s, b: (s, 0, 0)))
        args.append(tmat)
    out_specs = []
    out_shape = []
    if has_agg:
        out_specs.append(pl.BlockSpec((1, _BN, H), lambda s, b: (s, b, 0)))
        out_shape.append(jax.ShapeDtypeStruct((2, N, H), jnp.float32))
    if emit_y:
        out_specs.append(pl.BlockSpec((1, _BN, KCP), lambda s, b: (s, b, 0)))
        out_shape.append(jax.ShapeDtypeStruct((2, N, KCP), jnp.float32))

    def body(*refs):
        nin = len(args)
        i = 0
        agg_ref = None
        if has_agg:
            agg_ref = refs[i]; i += 1
        x_ref = refs[i]; i += 1
        root_ref = bias_ref = tm_ref = None
        if has_agg:
            root_ref = refs[i]; i += 1
            bias_ref = refs[i]; i += 1
        if emit_y:
            tm_ref = refs[i]; i += 1
        outs = refs[nin:]
        j = 0
        if has_agg:
            xn = (agg_ref[0]
                  + jnp.dot(x_ref[0], root_ref[0],
                            preferred_element_type=jnp.float32)
                  + bias_ref[0])
            if relu:
                xn = jnp.maximum(xn, 0.0)
            outs[j][0] = xn
            j += 1
        else:
            xn = x_ref[0]
        if emit_y:
            outs[j][0] = jnp.dot(xn, tm_ref[0], preferred_element_type=jnp.float32)

    return pl.pallas_call(
        body,
        grid=(2, nb),
        in_specs=in_specs,
        out_specs=out_specs,
        out_shape=out_shape,
    )(*args)


_BP = 2000


def _pool_side_body(x_ref, x3_ref, bat_ref, pw1_ref, pb1_ref, pw2_ref, pb2_ref,
                    att_ref, mean_ref, gmax_s, den_s, cnt_s, s3_s, num_s):
    p = pl.program_id(0)
    b = pl.program_id(1)
    nb = pl.num_programs(1)

    @pl.when(jnp.logical_and(p == 0, b == 0))
    def _init():
        gmax_s[...] = jnp.full((1, G), -jnp.inf, jnp.float32)
        den_s[...] = jnp.zeros((1, G), jnp.float32)
        cnt_s[...] = jnp.zeros((1, G), jnp.float32)
        s3_s[...] = jnp.zeros((G, H), jnp.float32)
        num_s[...] = jnp.zeros((G, D), jnp.float32)

    x = x_ref[...]
    bat = bat_ref[...]
    mask = (bat == lax.broadcasted_iota(jnp.int32, (1, G), 1)).astype(jnp.float32)
    g = (jnp.dot(jnp.maximum(jnp.dot(x, pw1_ref[...],
                                     preferred_element_type=jnp.float32)
                             + pb1_ref[...], 0.0),
                 pw2_ref[...], preferred_element_type=jnp.float32)
         + pb2_ref[...])

    @pl.when(p == 0)
    def _pass0():
        gm = jnp.max(jnp.where(mask > 0.0, g, -jnp.inf), axis=0, keepdims=True)
        gmax_s[...] = jnp.maximum(gmax_s[...], gm)
        cnt_s[...] = cnt_s[...] + jnp.sum(mask, axis=0, keepdims=True)
        s3_s[...] = s3_s[...] + lax.dot_general(
            mask, x3_ref[...], (((0,), (0,)), ((), ())),
            preferred_element_type=jnp.float32)

    @pl.when(p == 1)
    def _pass1():
        gb = jnp.sum(mask * gmax_s[...], axis=1, keepdims=True)
        ex = jnp.exp(g - gb)
        den_s[...] = den_s[...] + jnp.sum(mask * ex, axis=0, keepdims=True)
        num_s[...] = num_s[...] + lax.dot_general(
            mask, ex * x, (((0,), (0,)), ((), ())),
            preferred_element_type=jnp.float32)

    @pl.when(jnp.logical_and(p == 1, b == nb - 1))
    def _final():
        att_ref[...] = num_s[...] / den_s[...].reshape(G, 1)
        mean_ref[...] = s3_s[...] / jnp.maximum(cnt_s[...], 1.0).reshape(G, 1)


def _pool_side(x, x3, bat, pw1, pb1, pw2, pb2):
    nb = N // _BP
    return pl.pallas_call(
        _pool_side_body,
        grid=(2, nb),
        in_specs=[
            pl.BlockSpec((_BP, D), lambda p, b: (b, 0)),
            pl.BlockSpec((_BP, H), lambda p, b: (b, 0)),
            pl.BlockSpec((_BP, 1), lambda p, b: (b, 0)),
            pl.BlockSpec((D, D), lambda p, b: (0, 0)),
            pl.BlockSpec((1, D), lambda p, b: (0, 0)),
            pl.BlockSpec((D, 1), lambda p, b: (0, 0)),
            pl.BlockSpec((1, 1), lambda p, b: (0, 0)),
        ],
        out_specs=[
            pl.BlockSpec((G, D), lambda p, b: (0, 0)),
            pl.BlockSpec((G, H), lambda p, b: (0, 0)),
        ],
        out_shape=[jax.ShapeDtypeStruct((G, D), jnp.float32),
                   jax.ShapeDtypeStruct((G, H), jnp.float32)],
        scratch_shapes=[
            pltpu.VMEM((1, G), jnp.float32),
            pltpu.VMEM((1, G), jnp.float32),
            pltpu.VMEM((1, G), jnp.float32),
            pltpu.VMEM((G, H), jnp.float32),
            pltpu.VMEM((G, D), jnp.float32),
        ],
    )(x, x3, bat, pw1, pb1, pw2, pb2)


def _mlp_body(mp_ref, md_ref, ap_ref, ad_ref, l0w_ref, l0b_ref, l1w_ref,
              l1b_ref, out_ref):
    feats = jnp.concatenate([mp_ref[...], md_ref[...], ap_ref[...],
                             ad_ref[...]], axis=1)
    hh = jnp.maximum(jnp.dot(feats, l0w_ref[...],
                             preferred_element_type=jnp.float32)
                     + l0b_ref[...], 0.0)
    out_ref[...] = jnp.dot(hh, l1w_ref[...],
                           preferred_element_type=jnp.float32) + l1b_ref[...]


def _mlp(mp, md, ap, ad, l0w, l0b, l1w, l1b):
    return pl.pallas_call(
        _mlp_body,
        out_shape=jax.ShapeDtypeStruct((G, 1), jnp.float32),
    )(mp, md, ap, ad, l0w, l0b, l1w, l1b)


def _tmat(w2, b2, in_ch):
    t = w2.reshape(H, in_ch, H).transpose(1, 0, 2).reshape(in_ch, H * H)
    pad = jnp.zeros((in_ch, KCP - KC), jnp.float32)
    return jnp.concatenate([t, b2.reshape(in_ch, H), pad], axis=1)


def kernel(x_p, x_d, edge_attr_p, edge_attr_d, edge_index_p, edge_index_d,
           x_p_batch, x_d_batch, pool_W1, pool_b1, pool_W2, pool_b2,
           p_nn1_W1, p_nn1_b1, p_nn1_W2, p_nn1_b2,
           p_nn2_W1, p_nn2_b1, p_nn2_W2, p_nn2_b2,
           p_root0, p_bias0, p_root1, p_bias1, p_root2, p_bias2,
           d_nn1_W1, d_nn1_b1, d_nn1_W2, d_nn1_b2,
           d_nn2_W1, d_nn2_b1, d_nn2_W2, d_nn2_b2,
           d_root0, d_bias0, d_root1, d_bias1, d_root2, d_bias2,
           lin0_W, lin0_b, lin1_W, lin1_b):
    x2 = jnp.stack([x_p, x_d])
    ea2 = jnp.stack([edge_attr_p, edge_attr_d])
    src = jnp.concatenate([edge_index_p[0], edge_index_d[0] + N])
    dstf = jnp.concatenate([edge_index_p[1], edge_index_d[1] + N])
    dsth = jnp.concatenate([edge_index_p[1] // 4, 2560 + edge_index_d[1] // 4]).astype(jnp.int32).reshape(E2 // _BSC, _BSC)
    lo2 = (dstf % 4).astype(jnp.int32).reshape(2, E, 1)
    z128 = jnp.zeros((_BSC, 128), jnp.float32)
    rowids = (jnp.arange(16 * _ROWS_T, dtype=jnp.int32)
              .reshape(16 * _NCH, _BSC))

    w1a = jnp.stack([p_nn1_W1, d_nn1_W1])
    b1a = jnp.stack([p_nn1_b1, d_nn1_b1]).reshape(2, 1, H)
    w1b = jnp.stack([p_nn2_W1, d_nn2_W1])
    b1b = jnp.stack([p_nn2_b1, d_nn2_b1]).reshape(2, 1, H)
    h1, h2 = _edge_h(ea2, lo2, w1a, b1a, w1b, b1b)
    h1f = h1.reshape(E2, 32)
    h2f = h2.reshape(E2, 32)

    tm0 = jnp.stack([_tmat(p_nn1_W2, p_nn1_b2, D), _tmat(d_nn1_W2, d_nn1_b2, D)])
    tm2 = jnp.stack([_tmat(p_nn2_W2, p_nn2_b2, H), _tmat(d_nn2_W2, d_nn2_b2, H)])
    roots0 = jnp.stack([p_root0, d_root0])
    biases0 = jnp.stack([p_bias0, d_bias0]).reshape(2, 1, H)
    roots1 = jnp.stack([p_root1, d_root1])
    biases1 = jnp.stack([p_bias1, d_bias1]).reshape(2, 1, H)
    roots2 = jnp.stack([p_root2, d_root2])
    biases2 = jnp.stack([p_bias2, d_bias2]).reshape(2, 1, H)

    (y0,) = _layer(None, x2, None, None, tm0, relu=False, emit_y=True)
    agg0 = _unpack(_edge_step(y0.reshape(N2, KCP), h1f, src, dsth, z128, rowids))
    x1, y1 = _layer(agg0, x2, roots0, biases0, tm2, relu=True, emit_y=True)
    agg1 = _unpack(_edge_step(y1.reshape(N2, KCP), h2f, src, dsth, z128, rowids))
    x22, y2 = _layer(agg1, x1, roots1, biases1, tm2, relu=True, emit_y=True)
    agg2 = _unpack(_edge_step(y2.reshape(N2, KCP), h2f, src, dsth, z128, rowids))
    (x3,) = _layer(agg2, x22, roots2, biases2, None, relu=False, emit_y=False)

    ap, mp = _pool_side(x_p, x3[0], x_p_batch.reshape(N, 1), pool_W1,
                        pool_b1.reshape(1, D), pool_W2, pool_b2.reshape(1, 1))
    ad, md = _pool_side(x_d, x3[1], x_d_batch.reshape(N, 1), pool_W1,
                        pool_b1.reshape(1, D), pool_W2, pool_b2.reshape(1, 1))
    return _mlp(mp, md, ap, ad, lin0_W, lin0_b.reshape(1, H),
                lin1_W, lin1_b.reshape(1, 1))
